# baseline (speedup 1.0000x reference)
import functools

import jax
import jax.numpy as jnp
from jax import lax
from jax.experimental import pallas as pl
from jax.experimental.pallas import tpu as pltpu
from jax.experimental.pallas import tpu_sc as plsc

_SCALING = 2.0


def _fuse_body(wt_ref, at_ref, bt_ref, o_ref):
    eye = jnp.eye(64, dtype=jnp.float32)
    ft = wt_ref[...] + lax.dot(
        bt_ref[...], at_ref[...], preferred_element_type=jnp.float32
    ) * _SCALING
    t = lax.dot_general(
        ft, eye, (((0,), (0,)), ((), ())),
        preferred_element_type=jnp.float32,
    )
    o_ref[:, 0:64] = t


def _fused_table_pad(wT, aT, bT):
    D, V = wT.shape
    R = aT.shape[0]
    BLK = 8192
    nblk = (V + BLK - 1) // BLK
    return pl.pallas_call(
        _fuse_body,
        grid=(nblk,),
        in_specs=[
            pl.BlockSpec((D, BLK), lambda i: (0, i)),
            pl.BlockSpec((R, BLK), lambda i: (0, i)),
            pl.BlockSpec((D, R), lambda i: (0, 0)),
        ],
        out_specs=pl.BlockSpec((BLK, 2 * D), lambda i: (i, 0)),
        out_shape=jax.ShapeDtypeStruct((V, 2 * D), jnp.float32),
    )(wT, aT, bT)


def _sc_gather_t(table, idx2d):
    info = plsc.get_sparse_core_info()
    NC, NS = info.num_cores, info.num_subcores
    NW = NC * NS
    n_chunks, CH = idx2d.shape
    assert CH == 128 and n_chunks % NW == 0
    per_w = n_chunks // NW
    V, Dpad = table.shape
    D = 64

    mesh = plsc.VectorSubcoreMesh(core_axis_name="c", subcore_axis_name="s")

    @functools.partial(
        pl.kernel,
        mesh=mesh,
        compiler_params=pltpu.CompilerParams(
            use_tc_tiling_on_sc=False, needs_layout_passes=False
        ),
        out_type=jax.ShapeDtypeStruct((50, 8, 128, 1024), jnp.float32),
        scratch_types=[
            pltpu.VMEM((per_w, CH), jnp.int32),
            pltpu.VMEM((CH, Dpad), jnp.float32),
            pltpu.VMEM((8, 1024), jnp.float32),
            pltpu.SemaphoreType.DMA,
        ],
    )
    def k(table_hbm, idx_hbm, out_hbm, idx_v, rows_v, tbuf, sem):
        wid = lax.axis_index("s") * NC + lax.axis_index("c")
        chunk0 = wid * per_w
        pltpu.sync_copy(idx_hbm.at[pl.ds(chunk0, per_w)], idx_v)
        lane = lax.iota(jnp.int32, 16)

        def transpose_block(gi, carry):
            tok = 16 * gi + lane
            for d in range(D):
                vals = plsc.load_gather(
                    rows_v, [tok, jnp.full((16,), d, jnp.int32)]
                )
                tbuf[d // 8, pl.ds((d % 8) * 128 + 16 * gi, 16)] = vals
            return carry

        def body(j, carry):
            c = chunk0 + j
            l = c // 128
            tc = c % 128
            pltpu.async_copy(table_hbm.at[idx_v.at[j]], rows_v, sem).wait()
            lax.fori_loop(0, 8, transpose_block, 0)
            pltpu.sync_copy(tbuf, out_hbm.at[l, :, tc])
            return carry

        lax.fori_loop(0, per_w, body, 0)

    return k(table, idx2d)


def kernel(x, weight, lora_A, lora_B):
    B, H = x.shape
    V, D = weight.shape
    table = _fused_table_pad(weight.T, lora_A.T, lora_B.T)
    idx = x.astype(jnp.int32).T.reshape(-1, 128)
    out4 = _sc_gather_t(table, idx)
    out = (
        out4.reshape(H, 8, 128, 8, 128)
        .transpose(2, 4, 0, 1, 3)
        .reshape(B, H, D)
    )
    return out


# speedup vs baseline: 1.9404x; 1.9404x over previous
import functools

import jax
import jax.numpy as jnp
from jax import lax
from jax.experimental import pallas as pl
from jax.experimental.pallas import tpu as pltpu
from jax.experimental.pallas import tpu_sc as plsc

_SCALING = 2.0


def _fuse_body(wt_ref, at_ref, bt_ref, o_ref):
    eye = jnp.eye(64, dtype=jnp.float32)
    ft = wt_ref[...] + lax.dot(
        bt_ref[...], at_ref[...], preferred_element_type=jnp.float32
    ) * _SCALING
    t = lax.dot_general(
        ft, eye, (((0,), (0,)), ((), ())),
        preferred_element_type=jnp.float32,
    )
    o_ref[:, 0:64] = t


def _fused_table_pad(wT, aT, bT):
    D, V = wT.shape
    R = aT.shape[0]
    BLK = 8192
    nblk = (V + BLK - 1) // BLK
    return pl.pallas_call(
        _fuse_body,
        grid=(nblk,),
        in_specs=[
            pl.BlockSpec((D, BLK), lambda i: (0, i)),
            pl.BlockSpec((R, BLK), lambda i: (0, i)),
            pl.BlockSpec((D, R), lambda i: (0, 0)),
        ],
        out_specs=pl.BlockSpec((BLK, 2 * D), lambda i: (i, 0)),
        out_shape=jax.ShapeDtypeStruct((V, 2 * D), jnp.float32),
    )(wT, aT, bT)


def _sc_gather_perm(table, idx2d):
    info = plsc.get_sparse_core_info()
    NC, NS = info.num_cores, info.num_subcores
    NW = NC * NS
    n_chunks, CH = idx2d.shape
    assert CH == 128 and n_chunks % NW == 0
    per_w = n_chunks // NW
    V, Dpad = table.shape
    D = 64
    N = n_chunks * CH

    mesh = plsc.VectorSubcoreMesh(core_axis_name="c", subcore_axis_name="s")

    @functools.partial(
        pl.kernel,
        mesh=mesh,
        compiler_params=pltpu.CompilerParams(use_tc_tiling_on_sc=False),
        out_type=jax.ShapeDtypeStruct((N // 2, 2 * D), jnp.float32),
        scratch_types=[
            pltpu.VMEM((per_w, CH), jnp.int32),
            pltpu.VMEM((CH, Dpad), jnp.float32),
            pltpu.SemaphoreType.DMA,
        ],
    )
    def k(table_hbm, idx_hbm, out_hbm, idx_v, rows_v, sem):
        wid = lax.axis_index("s") * NC + lax.axis_index("c")
        chunk0 = wid * per_w
        pltpu.sync_copy(idx_hbm.at[pl.ds(chunk0, per_w)], idx_v)

        def body(j, carry):
            c = chunk0 + j
            g = c // 32
            k32 = c % 32
            h = k32 // 16
            q0 = g * 2048 + (k32 % 16) * 128
            pltpu.async_copy(table_hbm.at[idx_v.at[j]], rows_v, sem).wait()
            pltpu.sync_copy(
                rows_v.at[:, pl.ds(0, D)],
                out_hbm.at[pl.ds(q0, CH), pl.ds(h * D, D)],
            )
            return carry

        lax.fori_loop(0, per_w, body, 0)

    return k(table, idx2d)


def _transpose_body(in_ref, o_ref):
    eye = jnp.eye(64, dtype=jnp.float32)
    blk = in_ref[...]
    for h in range(2):
        t = lax.dot_general(
            eye, blk[:, h * 64:(h + 1) * 64], (((1,), (1,)), ((), ())),
            preferred_element_type=jnp.float32,
        )
        o_ref[0, :, h * 2048:(h + 1) * 2048] = t


def _tc_transpose(packed, L, B):
    NB = B // 4096
    return pl.pallas_call(
        _transpose_body,
        grid=(L, NB),
        in_specs=[pl.BlockSpec((2048, 128), lambda l, bb: (l * NB + bb, 0))],
        out_specs=pl.BlockSpec((1, 64, 4096), lambda l, bb: (l, 0, bb)),
        out_shape=jax.ShapeDtypeStruct((L, 64, B), jnp.float32),
    )(packed)


def kernel(x, weight, lora_A, lora_B):
    B, H = x.shape
    V, D = weight.shape
    table = _fused_table_pad(weight.T, lora_A.T, lora_B.T)
    idx = x.astype(jnp.int32).T.reshape(-1, 128)
    packed = _sc_gather_perm(table, idx)
    out3 = _tc_transpose(packed, H, B)
    return jnp.transpose(out3, (2, 0, 1))


# speedup vs baseline: 2.4886x; 1.2825x over previous
import functools

import jax
import jax.numpy as jnp
from jax import lax
from jax.experimental import pallas as pl
from jax.experimental.pallas import tpu as pltpu
from jax.experimental.pallas import tpu_sc as plsc

_SCALING = 2.0


def _fuse_body(wt_ref, at_ref, bt_ref, o_ref):
    eye = jnp.eye(64, dtype=jnp.float32)
    ft = wt_ref[...] + lax.dot(
        bt_ref[...], at_ref[...], preferred_element_type=jnp.float32
    ) * _SCALING
    t = lax.dot_general(
        ft, eye, (((0,), (0,)), ((), ())),
        preferred_element_type=jnp.float32,
    )
    o_ref[:, 0:64] = t


def _fused_table_pad(wT, aT, bT):
    D, V = wT.shape
    R = aT.shape[0]
    BLK = 8192
    nblk = (V + BLK - 1) // BLK
    return pl.pallas_call(
        _fuse_body,
        grid=(nblk,),
        in_specs=[
            pl.BlockSpec((D, BLK), lambda i: (0, i)),
            pl.BlockSpec((R, BLK), lambda i: (0, i)),
            pl.BlockSpec((D, R), lambda i: (0, 0)),
        ],
        out_specs=pl.BlockSpec((BLK, 2 * D), lambda i: (i, 0)),
        out_shape=jax.ShapeDtypeStruct((V, 2 * D), jnp.float32),
    )(wT, aT, bT)


def _sc_gather_perm(table, idx2d):
    info = plsc.get_sparse_core_info()
    NC, NS = info.num_cores, info.num_subcores
    NW = NC * NS
    n_chunks, CH = idx2d.shape
    assert CH == 128 and n_chunks % NW == 0
    per_w = n_chunks // NW
    V, Dpad = table.shape
    D = 64
    N = n_chunks * CH

    mesh = plsc.VectorSubcoreMesh(core_axis_name="c", subcore_axis_name="s")

    NBUF = 4
    assert per_w % NBUF == 0

    @functools.partial(
        pl.kernel,
        mesh=mesh,
        compiler_params=pltpu.CompilerParams(use_tc_tiling_on_sc=False),
        out_type=jax.ShapeDtypeStruct((N // 2, 2 * D), jnp.float32),
        scratch_types=[
            pltpu.VMEM((per_w, CH), jnp.int32),
            pltpu.VMEM((NBUF, CH, Dpad), jnp.float32),
            pltpu.SemaphoreType.DMA,
            pltpu.SemaphoreType.DMA,
            pltpu.SemaphoreType.DMA,
            pltpu.SemaphoreType.DMA,
        ],
    )
    def k(table_hbm, idx_hbm, out_hbm, idx_v, rows_v, s0, s1, s2, s3):
        sems = (s0, s1, s2, s3)
        wid = lax.axis_index("s") * NC + lax.axis_index("c")
        chunk0 = wid * per_w
        pltpu.sync_copy(idx_hbm.at[pl.ds(chunk0, per_w)], idx_v)

        def write_out(j, buf):
            c = chunk0 + j
            g = c // 32
            k32 = c % 32
            h = k32 // 16
            q0 = g * 2048 + (k32 % 16) * 128
            pltpu.sync_copy(
                buf.at[:, pl.ds(0, D)],
                out_hbm.at[pl.ds(q0, CH), pl.ds(h * D, D)],
            )

        for u in range(NBUF):
            pltpu.async_copy(table_hbm.at[idx_v.at[u]], rows_v.at[u], sems[u])

        def body(jj, carry):
            for u in range(NBUF):
                j = NBUF * jj + u
                pltpu.make_async_copy(
                    table_hbm.at[idx_v.at[j]], rows_v.at[u], sems[u]
                ).wait()
                write_out(j, rows_v.at[u])
                pltpu.async_copy(
                    table_hbm.at[idx_v.at[j + NBUF]], rows_v.at[u], sems[u]
                )
            return carry

        lax.fori_loop(0, per_w // NBUF - 1, body, 0)

        for u in range(NBUF):
            j = per_w - NBUF + u
            pltpu.make_async_copy(
                table_hbm.at[idx_v.at[j]], rows_v.at[u], sems[u]
            ).wait()
            write_out(j, rows_v.at[u])

    return k(table, idx2d)


def _transpose_body(in_ref, o_ref):
    eye = jnp.eye(64, dtype=jnp.float32)
    blk = in_ref[...]
    for h in range(2):
        t = lax.dot_general(
            eye, blk[:, h * 64:(h + 1) * 64], (((1,), (1,)), ((), ())),
            preferred_element_type=jnp.float32,
        )
        o_ref[0, :, h * 2048:(h + 1) * 2048] = t


def _tc_transpose(packed, L, B):
    NB = B // 4096
    return pl.pallas_call(
        _transpose_body,
        grid=(L, NB),
        in_specs=[pl.BlockSpec((2048, 128), lambda l, bb: (l * NB + bb, 0))],
        out_specs=pl.BlockSpec((1, 64, 4096), lambda l, bb: (l, 0, bb)),
        out_shape=jax.ShapeDtypeStruct((L, 64, B), jnp.float32),
    )(packed)


def kernel(x, weight, lora_A, lora_B):
    B, H = x.shape
    V, D = weight.shape
    table = _fused_table_pad(weight.T, lora_A.T, lora_B.T)
    idx = x.astype(jnp.int32).T.reshape(-1, 128)
    packed = _sc_gather_perm(table, idx)
    out3 = _tc_transpose(packed, H, B)
    return jnp.transpose(out3, (2, 0, 1))


# speedup vs baseline: 2.6858x; 1.0792x over previous
import functools

import jax
import jax.numpy as jnp
from jax import lax
from jax.experimental import pallas as pl
from jax.experimental.pallas import tpu as pltpu
from jax.experimental.pallas import tpu_sc as plsc

_SCALING = 2.0


def _fuse_body(wt_ref, at_ref, bt_ref, o_ref):
    eye = jnp.eye(64, dtype=jnp.float32)
    ft = wt_ref[...] + lax.dot(
        bt_ref[...], at_ref[...], preferred_element_type=jnp.float32
    ) * _SCALING
    t = lax.dot_general(
        ft, eye, (((0,), (0,)), ((), ())),
        preferred_element_type=jnp.float32,
    )
    o_ref[:, 0:64] = t


def _fused_table_pad(wT, aT, bT):
    D, V = wT.shape
    R = aT.shape[0]
    BLK = 8192
    nblk = (V + BLK - 1) // BLK
    return pl.pallas_call(
        _fuse_body,
        grid=(nblk,),
        in_specs=[
            pl.BlockSpec((D, BLK), lambda i: (0, i)),
            pl.BlockSpec((R, BLK), lambda i: (0, i)),
            pl.BlockSpec((D, R), lambda i: (0, 0)),
        ],
        out_specs=pl.BlockSpec((BLK, 2 * D), lambda i: (i, 0)),
        out_shape=jax.ShapeDtypeStruct((V, 2 * D), jnp.float32),
    )(wT, aT, bT)


def _sc_gather_perm(table, idx2d, ofs, n_chunks):
    info = plsc.get_sparse_core_info()
    NC, NS = info.num_cores, info.num_subcores
    NW = NC * NS
    CH = idx2d.shape[1]
    assert CH == 128 and n_chunks % NW == 0
    per_w = n_chunks // NW
    V, Dpad = table.shape
    D = 64
    N = n_chunks * CH

    mesh = plsc.VectorSubcoreMesh(core_axis_name="c", subcore_axis_name="s")

    NBUF = 4
    assert per_w % NBUF == 0

    @functools.partial(
        pl.kernel,
        mesh=mesh,
        compiler_params=pltpu.CompilerParams(use_tc_tiling_on_sc=False),
        out_type=jax.ShapeDtypeStruct((N // 2, 2 * D), jnp.float32),
        scratch_types=[
            pltpu.VMEM((per_w, CH), jnp.int32),
            pltpu.VMEM((NBUF, CH, Dpad), jnp.float32),
            pltpu.SemaphoreType.DMA,
            pltpu.SemaphoreType.DMA,
            pltpu.SemaphoreType.DMA,
            pltpu.SemaphoreType.DMA,
        ],
    )
    def k(table_hbm, idx_hbm, out_hbm, idx_v, rows_v, s0, s1, s2, s3):
        sems = (s0, s1, s2, s3)
        wid = lax.axis_index("s") * NC + lax.axis_index("c")
        chunk0 = wid * per_w
        pltpu.sync_copy(idx_hbm.at[pl.ds(ofs + chunk0, per_w)], idx_v)

        def write_out(j, buf):
            c = chunk0 + j
            g = c // 32
            k32 = c % 32
            h = k32 // 16
            q0 = g * 2048 + (k32 % 16) * 128
            pltpu.sync_copy(
                buf.at[:, pl.ds(0, D)],
                out_hbm.at[pl.ds(q0, CH), pl.ds(h * D, D)],
            )

        for u in range(NBUF):
            pltpu.async_copy(table_hbm.at[idx_v.at[u]], rows_v.at[u], sems[u])

        def body(jj, carry):
            for u in range(NBUF):
                j = NBUF * jj + u
                pltpu.make_async_copy(
                    table_hbm.at[idx_v.at[j]], rows_v.at[u], sems[u]
                ).wait()
                write_out(j, rows_v.at[u])
                pltpu.async_copy(
                    table_hbm.at[idx_v.at[j + NBUF]], rows_v.at[u], sems[u]
                )
            return carry

        lax.fori_loop(0, per_w // NBUF - 1, body, 0)

        for u in range(NBUF):
            j = per_w - NBUF + u
            pltpu.make_async_copy(
                table_hbm.at[idx_v.at[j]], rows_v.at[u], sems[u]
            ).wait()
            write_out(j, rows_v.at[u])

    return k(table, idx2d)


def _transpose_body(in_ref, o_ref):
    eye = jnp.eye(64, dtype=jnp.float32)
    blk = in_ref[...]
    for h in range(2):
        t = lax.dot_general(
            eye, blk[:, h * 64:(h + 1) * 64], (((1,), (1,)), ((), ())),
            preferred_element_type=jnp.float32,
        )
        o_ref[0, :, h * 2048:(h + 1) * 2048] = t


def _transpose_body_acc(in_ref, prev_ref, o_ref):
    del prev_ref
    _transpose_body(in_ref, o_ref)


def _tc_transpose_slice(packed, prev, lofs, Ls, L, B):
    NB = B // 4096
    in_specs = [pl.BlockSpec((2048, 128), lambda l, bb: (l * NB + bb, 0))]
    args = (packed,)
    aliases = {}
    body = _transpose_body
    if prev is not None:
        in_specs.append(pl.BlockSpec(memory_space=pl.ANY))
        args = (packed, prev)
        aliases = {1: 0}
        body = _transpose_body_acc
    return pl.pallas_call(
        body,
        grid=(Ls, NB),
        in_specs=in_specs,
        out_specs=pl.BlockSpec(
            (1, 64, 4096), lambda l, bb, lofs=lofs: (lofs + l, 0, bb)
        ),
        out_shape=jax.ShapeDtypeStruct((L, 64, B), jnp.float32),
        input_output_aliases=aliases,
    )(*args)


def kernel(x, weight, lora_A, lora_B):
    B, H = x.shape
    V, D = weight.shape
    table = _fused_table_pad(weight.T, lora_A.T, lora_B.T)
    idx = x.astype(jnp.int32).T.reshape(-1, 128)
    SLICES = 5
    Ls = H // SLICES
    n_chunks = idx.shape[0]
    cps = n_chunks // SLICES
    out3 = None
    for s in range(SLICES):
        packed = _sc_gather_perm(table, idx, s * cps, cps)
        out3 = _tc_transpose_slice(packed, out3, s * Ls, Ls, H, B)
    return jnp.transpose(out3, (2, 0, 1))
